# baseline (speedup 1.0000x reference)
import functools
import jax
import jax.numpy as jnp
from jax import lax
from jax.experimental import pallas as pl
from jax.experimental.pallas import tpu as pltpu
from jax.experimental.pallas import tpu_sc as plsc

D = 64
ROWS = 16384
COLS = 20
B = ROWS * COLS
NC, NS, L = 2, 16, 16
NW = NC * NS
BPW = B // NW
C = 512
NCHUNK = BPW // C
SCALE = 8.0

_mesh = plsc.VectorSubcoreMesh(core_axis_name="c", subcore_axis_name="s")


@functools.partial(
    pl.kernel,
    out_type=jax.ShapeDtypeStruct((B, D), jnp.float32),
    mesh=_mesh,
    scratch_types=[
        pltpu.VMEM((BPW,), jnp.int32),
        pltpu.VMEM((C, D), jnp.float32),
        pltpu.SemaphoreType.DMA,
    ],
    compiler_params=pltpu.CompilerParams(use_tc_tiling_on_sc=False),
)
def _emb(x_hbm, table_hbm, out_hbm, idx_v, rows_v, sem):
    wid = lax.axis_index("s") * NC + lax.axis_index("c")
    base = wid * BPW
    pltpu.sync_copy(x_hbm.at[pl.ds(base, BPW)], idx_v)

    def chunk_body(g, _):
        off = pl.multiple_of(g * C, C)
        pltpu.async_copy(
            table_hbm.at[idx_v.at[pl.ds(off, C)]], rows_v, sem
        ).wait()

        def scale_row(r, _):
            for j in range(D // L):
                sl = pl.ds(j * L, L)
                rows_v[r, sl] = rows_v[r, sl] * SCALE
            return 0

        lax.fori_loop(0, C, scale_row, 0)
        pltpu.sync_copy(rows_v, out_hbm.at[pl.ds(base + off, C)])
        return 0

    lax.fori_loop(0, NCHUNK, chunk_body, 0)


def kernel(x, table):
    x_flat = x.reshape(B).astype(jnp.int32)
    out = _emb(x_flat, table)
    return out.reshape(ROWS, COLS, D)


# speedup vs baseline: 1.0199x; 1.0199x over previous
import functools
import jax
import jax.numpy as jnp
from jax import lax
from jax.experimental import pallas as pl
from jax.experimental.pallas import tpu as pltpu
from jax.experimental.pallas import tpu_sc as plsc

D = 64
ROWS = 16384
COLS = 20
NC, NS, L = 2, 16, 16
NW = NC * NS
RPW = ROWS // NW
BPW = RPW * COLS
XC = 32
C = XC * COLS
NCHUNK = RPW // XC
SCALE = 8.0

_mesh = plsc.VectorSubcoreMesh(core_axis_name="c", subcore_axis_name="s")


@functools.partial(
    pl.kernel,
    out_type=jax.ShapeDtypeStruct((ROWS, COLS, D), jnp.float32),
    mesh=_mesh,
    scratch_types=[
        pltpu.VMEM((RPW, COLS), jnp.int32),
        pltpu.VMEM((BPW,), jnp.int32),
        pltpu.VMEM((C, D), jnp.float32),
        pltpu.VMEM((XC, COLS, D), jnp.float32),
        pltpu.SemaphoreType.DMA,
    ],
    compiler_params=pltpu.CompilerParams(
        use_tc_tiling_on_sc=False, needs_layout_passes=False
    ),
)
def _emb(x_hbm, table_hbm, out_hbm, idx_c, idx_v, rows_v, out_c, sem):
    wid = lax.axis_index("s") * NC + lax.axis_index("c")
    pltpu.sync_copy(x_hbm.at[pl.ds(wid * RPW, RPW)], idx_c)

    lane = lax.iota(jnp.int32, L)

    def flatten_body(m, _):
        for j in range(5):
            g = lane + j * L
            rj = lax.shift_right_logical(g * 3277, 16)
            cj = g - COLS * rj
            v = plsc.load_gather(idx_c, [m * 4 + rj, cj])
            idx_v[pl.ds(m * 80 + j * L, L)] = v
        return 0

    lax.fori_loop(0, RPW // 4, flatten_body, 0)

    def chunk_body(g, _):
        off = pl.multiple_of(g * C, C)
        pltpu.async_copy(
            table_hbm.at[idx_v.at[pl.ds(off, C)]], rows_v, sem
        ).wait()

        def scale_row(q, _):
            for c in range(COLS):
                for j in range(D // L):
                    sl = pl.ds(j * L, L)
                    out_c[q, c, sl] = rows_v[q * COLS + c, sl] * SCALE
            return 0

        lax.fori_loop(0, XC, scale_row, 0)
        pltpu.sync_copy(out_c, out_hbm.at[pl.ds(wid * RPW + g * XC, XC)])
        return 0

    lax.fori_loop(0, NCHUNK, chunk_body, 0)


def kernel(x, table):
    return _emb(x.astype(jnp.int32), table)
